# baseline (speedup 1.0000x reference)
import functools

import jax
import jax.numpy as jnp
from jax import lax
from jax.experimental import pallas as pl
from jax.experimental.pallas import tpu as pltpu
from jax.experimental.pallas import tpu_sc as plsc

NC = 2
NS = 16
NW = NC * NS


def _sc_gather_build(tokens, V, D):
    tpw = tokens // NW
    ck = 64
    nchunks = tpw // ck
    mesh = plsc.VectorSubcoreMesh(core_axis_name="c", subcore_axis_name="s")

    @functools.partial(
        pl.kernel,
        mesh=mesh,
        out_type=jax.ShapeDtypeStruct((tokens, D), jnp.float32),
        scratch_types=[
            pltpu.VMEM((ck,), jnp.int32),
            pltpu.VMEM((ck, D), jnp.float32),
            pltpu.SemaphoreType.DMA,
        ],
        compiler_params=pltpu.CompilerParams(use_tc_tiling_on_sc=True),
    )
    def sc_gather(ids_hbm, table_hbm, e_hbm, idx_v, rows_v, sem):
        wid = lax.axis_index("s") * NC + lax.axis_index("c")
        base = wid * tpw
        for k in range(nchunks):
            pltpu.sync_copy(ids_hbm.at[pl.ds(base + k * ck, ck)], idx_v)
            pltpu.async_copy(table_hbm.at[idx_v], rows_v, sem).wait()
            pltpu.sync_copy(rows_v, e_hbm.at[pl.ds(base + k * ck, ck)])

    return sc_gather


def _fused_build(B, S, P, D, tile):
    nj = (P + S + tile - 1) // tile

    def body(e_ref, sp_ref, w_ref, b_ref, m_ref, o_ref, tail_ref):
        j = pl.program_id(1)
        body_rows = e_ref[0, : tile - P, :]
        tail = jnp.where(j == 0, sp_ref[:P, :], tail_ref[:P, :])
        window = jnp.concatenate([tail, body_rows], axis=0)
        acc = jnp.dot(window, w_ref[...], preferred_element_type=jnp.float32)
        o_ref[0] = jnp.tanh(acc + b_ref[...]) * m_ref[0]
        tail_ref[:P, :] = e_ref[0, tile - P : tile, :]

    grid = (B, nj)
    return pl.pallas_call(
        body,
        grid=grid,
        in_specs=[
            pl.BlockSpec((1, tile, D),
                         lambda b, j: (b, jnp.minimum(j, S // tile - 1), 0)),
            pl.BlockSpec((8 * ((P + 7) // 8), D), lambda b, j: (0, 0)),
            pl.BlockSpec((D, D), lambda b, j: (0, 0)),
            pl.BlockSpec((1, D), lambda b, j: (0, 0)),
            pl.BlockSpec((1, tile, 1), lambda b, j: (b, j, 0)),
        ],
        out_specs=pl.BlockSpec((1, tile, D), lambda b, j: (b, j, 0)),
        out_shape=jax.ShapeDtypeStruct((B, P + S, D), jnp.float32),
        scratch_shapes=[pltpu.VMEM((8 * ((P + 7) // 8), D), jnp.float32)],
        compiler_params=pltpu.CompilerParams(
            dimension_semantics=("arbitrary", "arbitrary"),
        ),
    )


def kernel(input_ids, attention_mask, token_type_ids, word_embeddings,
           soft_prompt, W, b):
    B, S = input_ids.shape
    V, D = word_embeddings.shape
    P = soft_prompt.shape[0]

    ids = input_ids.reshape(-1).astype(jnp.int32)
    sc_gather = _sc_gather_build(B * S, V, D)
    e = sc_gather(ids, word_embeddings).reshape(B, S, D)

    p8 = 8 * ((P + 7) // 8)
    sp_pad = jnp.pad(soft_prompt, ((0, p8 - P), (0, 0)))
    mask = jnp.concatenate(
        [jnp.ones((B, P), dtype=attention_mask.dtype), attention_mask], axis=1
    ).reshape(B, P + S, 1).astype(jnp.float32)

    fused = _fused_build(B, S, P, D, 512)
    return fused(e, sp_pad, W, b.reshape(1, D), mask)


# speedup vs baseline: 1.1345x; 1.1345x over previous
import functools

import jax
import jax.numpy as jnp
from jax import lax
from jax.experimental import pallas as pl
from jax.experimental.pallas import tpu as pltpu
from jax.experimental.pallas import tpu_sc as plsc

NC = 2
NS = 16
NW = NC * NS


def _sc_gather_build(B, S, P, V, D):
    tokens = B * S
    tpw = tokens // NW
    ck = 64
    nchunks = tpw // ck
    wpb = NW // B
    p8 = 8 * ((P + 7) // 8)
    mesh = plsc.VectorSubcoreMesh(core_axis_name="c", subcore_axis_name="s")

    @functools.partial(
        pl.kernel,
        mesh=mesh,
        out_type=jax.ShapeDtypeStruct(((P + S) * B, D), jnp.float32),
        scratch_types=[
            pltpu.VMEM((ck,), jnp.int32),
            pltpu.VMEM((ck,), jnp.int32),
            pltpu.VMEM((ck, D), jnp.float32),
            pltpu.VMEM((p8,), jnp.int32),
            pltpu.VMEM((p8, D), jnp.float32),
            pltpu.SemaphoreType.DMA,
        ],
        compiler_params=pltpu.CompilerParams(use_tc_tiling_on_sc=True),
    )
    def sc_gather(ids_hbm, table_hbm, sp_hbm, dpos_hbm, pidx_hbm, x_hbm,
                  idx_v, didx_v, rows_v, pidx_v, sp_v, sem):
        wid = lax.axis_index("s") * NC + lax.axis_index("c")
        batch = wid // wpb
        sub = wid % wpb

        @pl.when(sub == 0)
        def _():
            pltpu.sync_copy(sp_hbm, sp_v)
            pltpu.sync_copy(pidx_hbm.at[batch], pidx_v)
            pltpu.async_copy(sp_v, x_hbm.at[pidx_v], sem).wait()

        src_base = wid * tpw
        for k in range(nchunks):
            pltpu.sync_copy(ids_hbm.at[pl.ds(src_base + k * ck, ck)], idx_v)
            pltpu.sync_copy(dpos_hbm.at[pl.ds(src_base + k * ck, ck)], didx_v)
            pltpu.async_copy(table_hbm.at[idx_v], rows_v, sem).wait()
            pltpu.async_copy(rows_v, x_hbm.at[didx_v], sem).wait()

    return sc_gather


def _tc_body(x_ref, w_ref, b_ref, m_ref, o_ref):
    acc = jnp.dot(x_ref[...], w_ref[...], preferred_element_type=jnp.float32)
    o_ref[...] = jnp.tanh(acc + b_ref[...]) * m_ref[...]


def _tc_matmul_build(rows, D, tile):
    nj = (rows + tile - 1) // tile
    return pl.pallas_call(
        _tc_body,
        grid=(nj,),
        in_specs=[
            pl.BlockSpec((tile, D), lambda j: (j, 0)),
            pl.BlockSpec((D, D), lambda j: (0, 0)),
            pl.BlockSpec((1, D), lambda j: (0, 0)),
            pl.BlockSpec((tile, 1), lambda j: (j, 0)),
        ],
        out_specs=pl.BlockSpec((tile, D), lambda j: (j, 0)),
        out_shape=jax.ShapeDtypeStruct((rows, D), jnp.float32),
        compiler_params=pltpu.CompilerParams(
            dimension_semantics=("arbitrary",),
        ),
    )


def kernel(input_ids, attention_mask, token_type_ids, word_embeddings,
           soft_prompt, W, b):
    B, S = input_ids.shape
    V, D = word_embeddings.shape
    P = soft_prompt.shape[0]
    p8 = 8 * ((P + 7) // 8)

    ids = input_ids.reshape(-1).astype(jnp.int32)
    sp_pad = jnp.pad(soft_prompt, ((0, p8 - P), (0, 0)))
    dpos = ((P + jnp.arange(S, dtype=jnp.int32))[None, :] * B
            + jnp.arange(B, dtype=jnp.int32)[:, None]).reshape(-1)
    pidx = (jnp.arange(p8, dtype=jnp.int32)[None, :] * B
            + jnp.arange(B, dtype=jnp.int32)[:, None])

    sc_gather = _sc_gather_build(B, S, P, V, D)
    x = sc_gather(ids, word_embeddings, sp_pad, dpos, pidx)

    mask = jnp.concatenate(
        [jnp.ones((B, P), dtype=attention_mask.dtype), attention_mask], axis=1
    ).astype(jnp.float32).T.reshape((P + S) * B, 1)

    tc = _tc_matmul_build((P + S) * B, D, 512)
    out_flat = tc(x, W, b.reshape(1, D), mask)
    return out_flat.reshape(P + S, B, D).transpose(1, 0, 2)


# speedup vs baseline: 1.5020x; 1.3240x over previous
import functools

import jax
import jax.numpy as jnp
from jax import lax
from jax.experimental import pallas as pl
from jax.experimental.pallas import tpu as pltpu
from jax.experimental.pallas import tpu_sc as plsc

NC = 2
NS = 16
NW = NC * NS


def _sc_gather_build(B, S, P, V, D):
    tokens = B * S
    tpw = tokens // NW
    ck = 64
    nchunks = tpw // ck
    wpb = NW // B
    p8 = 8 * ((P + 7) // 8)
    mesh = plsc.VectorSubcoreMesh(core_axis_name="c", subcore_axis_name="s")

    @functools.partial(
        pl.kernel,
        mesh=mesh,
        out_type=jax.ShapeDtypeStruct(((P + S) * B, D), jnp.float32),
        scratch_types=[
            pltpu.VMEM((ck,), jnp.int32),
            pltpu.VMEM((ck,), jnp.int32),
            pltpu.VMEM((ck, D), jnp.float32),
            pltpu.VMEM((p8,), jnp.int32),
            pltpu.VMEM((p8, D), jnp.float32),
            pltpu.SemaphoreType.DMA,
        ],
        compiler_params=pltpu.CompilerParams(use_tc_tiling_on_sc=True),
    )
    def sc_gather(ids_hbm, table_hbm, sp_hbm, dpos_hbm, pidx_hbm, x_hbm,
                  idx_v, didx_v, rows_v, pidx_v, sp_v, sem):
        wid = lax.axis_index("s") * NC + lax.axis_index("c")
        batch = wid // wpb
        sub = wid % wpb

        @pl.when(sub == 0)
        def _():
            pltpu.sync_copy(sp_hbm, sp_v)
            pltpu.sync_copy(pidx_hbm.at[batch], pidx_v)
            pltpu.async_copy(sp_v, x_hbm.at[pidx_v], sem).wait()

        src_base = wid * tpw
        for k in range(nchunks):
            pltpu.sync_copy(ids_hbm.at[pl.ds(src_base + k * ck, ck)], idx_v)
            pltpu.sync_copy(dpos_hbm.at[pl.ds(src_base + k * ck, ck)], didx_v)
            pltpu.async_copy(table_hbm.at[idx_v], rows_v, sem).wait()
            pltpu.async_copy(rows_v, x_hbm.at[didx_v], sem).wait()

    return sc_gather


def _tc_matmul_build(B, R, D, tile):
    lane = 128
    nt = D // lane
    rt = tile // B
    nj = (R + rt - 1) // rt

    def body(x_ref, w_ref, b_ref, m_ref, o_ref):
        acc = jnp.dot(x_ref[...], w_ref[...],
                      preferred_element_type=jnp.float32)
        h = jnp.tanh(acc + b_ref[...]) * m_ref[...]
        o_ref[...] = h.reshape(rt, B, nt, lane).transpose(0, 2, 1, 3).reshape(
            rt, nt * B, lane)

    return pl.pallas_call(
        body,
        grid=(nj,),
        in_specs=[
            pl.BlockSpec((tile, D), lambda j: (j, 0)),
            pl.BlockSpec((D, D), lambda j: (0, 0)),
            pl.BlockSpec((1, D), lambda j: (0, 0)),
            pl.BlockSpec((tile, 1), lambda j: (j, 0)),
        ],
        out_specs=pl.BlockSpec((rt, nt * B, lane), lambda j: (j, 0, 0)),
        out_shape=jax.ShapeDtypeStruct((R, nt * B, lane), jnp.float32),
        compiler_params=pltpu.CompilerParams(
            dimension_semantics=("arbitrary",),
        ),
    )


def kernel(input_ids, attention_mask, token_type_ids, word_embeddings,
           soft_prompt, W, b):
    B, S = input_ids.shape
    V, D = word_embeddings.shape
    P = soft_prompt.shape[0]
    p8 = 8 * ((P + 7) // 8)

    ids = input_ids.reshape(-1).astype(jnp.int32)
    sp_pad = jnp.pad(soft_prompt, ((0, p8 - P), (0, 0)))
    dpos = ((P + jnp.arange(S, dtype=jnp.int32))[None, :] * B
            + jnp.arange(B, dtype=jnp.int32)[:, None]).reshape(-1)
    pidx = (jnp.arange(p8, dtype=jnp.int32)[None, :] * B
            + jnp.arange(B, dtype=jnp.int32)[:, None])

    sc_gather = _sc_gather_build(B, S, P, V, D)
    x = sc_gather(ids, word_embeddings, sp_pad, dpos, pidx)

    mask = jnp.concatenate(
        [jnp.ones((B, P), dtype=attention_mask.dtype), attention_mask], axis=1
    ).astype(jnp.float32).T.reshape((P + S) * B, 1)

    tc = _tc_matmul_build(B, P + S, D, 512)
    out3 = tc(x, W, b.reshape(1, D), mask)
    lane = 128
    return (out3.reshape(P + S, D // lane, B, lane)
            .transpose(2, 0, 1, 3).reshape(B, P + S, D))


# speedup vs baseline: 1.5040x; 1.0013x over previous
import functools

import jax
import jax.numpy as jnp
from jax import lax
from jax.experimental import pallas as pl
from jax.experimental.pallas import tpu as pltpu
from jax.experimental.pallas import tpu_sc as plsc

NC = 2
NS = 16
NW = NC * NS


def _sc_gather_build(B, S, P, V, D):
    tokens = B * S
    tpw = tokens // NW
    ck = 64
    nchunks = tpw // ck
    wpb = NW // B
    p8 = 8 * ((P + 7) // 8)
    mesh = plsc.VectorSubcoreMesh(core_axis_name="c", subcore_axis_name="s")

    @functools.partial(
        pl.kernel,
        mesh=mesh,
        out_type=jax.ShapeDtypeStruct(((P + S) * B, D), jnp.float32),
        scratch_types=[
            pltpu.VMEM((ck,), jnp.int32),
            pltpu.VMEM((ck,), jnp.int32),
            pltpu.VMEM((ck, D), jnp.float32),
            pltpu.VMEM((p8,), jnp.int32),
            pltpu.VMEM((p8, D), jnp.float32),
            pltpu.SemaphoreType.DMA,
        ],
        compiler_params=pltpu.CompilerParams(use_tc_tiling_on_sc=True),
    )
    def sc_gather(ids_hbm, table_hbm, sp_hbm, dpos_hbm, pidx_hbm, x_hbm,
                  idx_v, didx_v, rows_v, pidx_v, sp_v, sem):
        wid = lax.axis_index("s") * NC + lax.axis_index("c")
        batch = wid // wpb
        sub = wid % wpb

        @pl.when(sub == 0)
        def _():
            pltpu.sync_copy(sp_hbm, sp_v)
            pltpu.sync_copy(pidx_hbm.at[batch], pidx_v)
            pltpu.async_copy(sp_v, x_hbm.at[pidx_v], sem).wait()

        src_base = wid * tpw
        for k in range(nchunks):
            pltpu.sync_copy(ids_hbm.at[pl.ds(src_base + k * ck, ck)], idx_v)
            pltpu.sync_copy(dpos_hbm.at[pl.ds(src_base + k * ck, ck)], didx_v)
            pltpu.async_copy(table_hbm.at[idx_v], rows_v, sem).wait()
            pltpu.async_copy(rows_v, x_hbm.at[didx_v], sem).wait()

    return sc_gather


def _tc_matmul_build(B, R, D, tile):
    lane = 128
    nt = D // lane
    rt = tile // B
    nj = (R + rt - 1) // rt

    def body(x_ref, w_ref, b_ref, m_ref, o_ref):
        acc = jnp.dot(x_ref[...].astype(jnp.bfloat16),
                      w_ref[...].astype(jnp.bfloat16),
                      preferred_element_type=jnp.float32)
        h = jnp.tanh(acc + b_ref[...]) * m_ref[...]
        o_ref[...] = h.reshape(rt, B, nt, lane).transpose(0, 2, 1, 3).reshape(
            rt, nt * B, lane)

    return pl.pallas_call(
        body,
        grid=(nj,),
        in_specs=[
            pl.BlockSpec((tile, D), lambda j: (j, 0)),
            pl.BlockSpec((D, D), lambda j: (0, 0)),
            pl.BlockSpec((1, D), lambda j: (0, 0)),
            pl.BlockSpec((tile, 1), lambda j: (j, 0)),
        ],
        out_specs=pl.BlockSpec((rt, nt * B, lane), lambda j: (j, 0, 0)),
        out_shape=jax.ShapeDtypeStruct((R, nt * B, lane), jnp.float32),
        compiler_params=pltpu.CompilerParams(
            dimension_semantics=("arbitrary",),
        ),
    )


def kernel(input_ids, attention_mask, token_type_ids, word_embeddings,
           soft_prompt, W, b):
    B, S = input_ids.shape
    V, D = word_embeddings.shape
    P = soft_prompt.shape[0]
    p8 = 8 * ((P + 7) // 8)

    ids = input_ids.reshape(-1).astype(jnp.int32)
    sp_pad = jnp.pad(soft_prompt, ((0, p8 - P), (0, 0)))
    dpos = ((P + jnp.arange(S, dtype=jnp.int32))[None, :] * B
            + jnp.arange(B, dtype=jnp.int32)[:, None]).reshape(-1)
    pidx = (jnp.arange(p8, dtype=jnp.int32)[None, :] * B
            + jnp.arange(B, dtype=jnp.int32)[:, None])

    sc_gather = _sc_gather_build(B, S, P, V, D)
    x = sc_gather(ids, word_embeddings, sp_pad, dpos, pidx)

    mask = jnp.concatenate(
        [jnp.ones((B, P), dtype=attention_mask.dtype), attention_mask], axis=1
    ).astype(jnp.float32).T.reshape((P + S) * B, 1)

    tc = _tc_matmul_build(B, P + S, D, 512)
    out3 = tc(x, W, b.reshape(1, D), mask)
    lane = 128
    return (out3.reshape(P + S, D // lane, B, lane)
            .transpose(2, 0, 1, 3).reshape(B, P + S, D))
